# baseline (speedup 1.0000x reference)
import functools

import jax
import jax.numpy as jnp
from jax import lax
from jax.experimental import pallas as pl
from jax.experimental.pallas import tpu as pltpu
from jax.experimental.pallas import tpu_sc as plsc

N_ROWS = 32768
D = 512
NSEG = 16
LANES = 16
NC = 2
NS = 16
NW = NC * NS
CW = 128
NCB = D // CW
NRG = NW // NCB
RPG = N_ROWS // NRG
CHUNK = 256
NCHUNK = RPG // CHUNK
NPAIR = NCHUNK // 2
NV = CW // LANES
SEGR = NSEG // 2

_mesh = plsc.VectorSubcoreMesh(
    core_axis_name="c", subcore_axis_name="s", num_cores=NC, num_subcores=NS
)


@functools.partial(
    pl.kernel,
    out_type=(
        jax.ShapeDtypeStruct((NSEG, 2 * D), jnp.float32),
        jax.ShapeDtypeStruct((NW, NSEG, CW), jnp.float32),
        jax.ShapeDtypeStruct((NW, NSEG, CW), jnp.float32),
        jax.ShapeDtypeStruct((NW, LANES), jnp.float32),
    ),
    mesh=_mesh,
    scratch_types=[
        pltpu.VMEM((RPG,), jnp.int32),
        pltpu.VMEM((CHUNK, CW), jnp.float32),
        pltpu.VMEM((CHUNK, CW), jnp.float32),
        pltpu.SemaphoreType.DMA,
        pltpu.SemaphoreType.DMA,
        pltpu.VMEM((NSEG, CW), jnp.float32),
        pltpu.VMEM((NSEG, CW), jnp.float32),
        pltpu.VMEM((LANES,), jnp.float32),
        pltpu.VMEM((NRG, LANES), jnp.float32),
        pltpu.VMEM((SEGR, CW), jnp.float32),
        pltpu.VMEM((SEGR, CW), jnp.float32),
    ],
    compiler_params=pltpu.CompilerParams(needs_layout_passes=False),
)
def _seg_meanmax(feat_hbm, idx_hbm, out_hbm, psum_hbm, pmax_hbm, pcnt_hbm,
                 idx_v, fbuf0, fbuf1, sem0, sem1, sum_v, max_v, cnt_v, cbuf,
                 rbuf_s, rbuf_m):
    cid = lax.axis_index("c")
    sid = lax.axis_index("s")
    grp = sid // NRG
    rg = sid % NRG
    cb = cid * (NS // NRG) + grp
    col0 = cb * CW
    rbase = rg * RPG
    wid = cid * NS + sid

    pltpu.sync_copy(idx_hbm.at[pl.ds(rbase, RPG)], idx_v)

    def lower_bound(targets):
        lo = jnp.zeros((LANES,), jnp.int32)
        hi = jnp.full((LANES,), RPG, jnp.int32)

        def step(_, c):
            l, h = c
            mid = lax.div(l + h, 2)
            vals = plsc.load_gather(idx_v, [mid])
            pred = vals < targets
            return (jnp.where(pred, mid + 1, l), jnp.where(pred, h, mid))

        lo, hi = lax.fori_loop(0, 12, step, (lo, hi))
        return lo

    seg_ids = lax.iota(jnp.int32, LANES)
    blo = lower_bound(seg_ids)
    bhi = lower_bound(seg_ids + 1)
    cnt_v[...] = (bhi - blo).astype(jnp.float32)

    zero = jnp.zeros((LANES,), jnp.float32)
    ninf = jnp.full((LANES,), -jnp.inf, jnp.float32)
    for si in range(NSEG):
        for j in range(NV):
            sum_v[si, pl.ds(LANES * j, LANES)] = zero
            max_v[si, pl.ds(LANES * j, LANES)] = ninf

    def copy_of(ci, buf, sem):
        roff = ci * CHUNK
        return pltpu.make_async_copy(
            feat_hbm.at[pl.ds(rbase + roff, CHUNK), pl.ds(col0, CW)], buf, sem
        )

    def process(fbuf, ci):
        roff = ci * CHUNK
        for si in range(NSEG):
            lo = jnp.maximum(blo[si], roff)
            hi = jnp.minimum(bhi[si], roff + CHUNK)

            @plsc.parallel_loop(
                lo, hi, step=1, unroll=2,
                carry=((zero,) * NV, (ninf,) * NV),
            )
            def run_loop(i, c, roff=roff):
                ss, mm = c
                b = i - roff
                new_ss = []
                new_mm = []
                for j in range(NV):
                    v = fbuf[b, pl.ds(LANES * j, LANES)]
                    new_ss.append(ss[j] + v)
                    new_mm.append(jnp.maximum(mm[j], v))
                return (tuple(new_ss), tuple(new_mm))

            ss, mm = run_loop
            for j in range(NV):
                sl = pl.ds(LANES * j, LANES)
                sum_v[si, sl] = sum_v[si, sl] + ss[j]
                max_v[si, sl] = jnp.maximum(max_v[si, sl], mm[j])

    copy_of(0, fbuf0, sem0).start()

    def pair_body(q, _):
        c0 = 2 * q
        copy_of(c0, fbuf0, sem0).wait()
        copy_of(c0 + 1, fbuf1, sem1).start()
        process(fbuf0, c0)
        copy_of(c0 + 1, fbuf1, sem1).wait()

        @pl.when(q < NPAIR - 1)
        def _prefetch():
            copy_of(c0 + 2, fbuf0, sem0).start()

        process(fbuf1, c0 + 1)
        return 0

    lax.fori_loop(0, NPAIR, pair_body, 0)

    pltpu.sync_copy(sum_v, psum_hbm.at[wid])
    pltpu.sync_copy(max_v, pmax_hbm.at[wid])
    pltpu.sync_copy(cnt_v, pcnt_hbm.at[wid])
    plsc.subcore_barrier()

    def _reduce(sg0):
        src0 = cid * NS + grp * NRG

        pltpu.sync_copy(pcnt_hbm.at[pl.ds(src0, NRG)], cbuf)
        counts = cbuf[0, :]
        for k in range(1, NRG):
            counts = counts + cbuf[k, :]
        recip = 1.0 / jnp.maximum(counts, 1.0)

        for si in range(SEGR):
            for j in range(NV):
                sl = pl.ds(LANES * j, LANES)
                sum_v[sg0 + si, sl] = zero
                max_v[sg0 + si, sl] = ninf

        def fold(k, _):
            pltpu.sync_copy(psum_hbm.at[src0 + k, pl.ds(sg0, SEGR)], rbuf_s)
            pltpu.sync_copy(pmax_hbm.at[src0 + k, pl.ds(sg0, SEGR)], rbuf_m)
            for si in range(SEGR):
                for j in range(NV):
                    sl = pl.ds(LANES * j, LANES)
                    sum_v[sg0 + si, sl] = sum_v[sg0 + si, sl] + rbuf_s[si, sl]
                    max_v[sg0 + si, sl] = jnp.maximum(
                        max_v[sg0 + si, sl], rbuf_m[si, sl]
                    )
            return 0

        lax.fori_loop(0, NRG, fold, 0)

        for si in range(SEGR):
            seg = sg0 + si
            has = counts[seg] > 0.0
            r = recip[seg]
            for j in range(NV):
                sl = pl.ds(LANES * j, LANES)
                sum_v[seg, sl] = jnp.where(has, sum_v[seg, sl] * r, zero)
                max_v[seg, sl] = jnp.where(has, max_v[seg, sl], zero)

        pltpu.sync_copy(
            sum_v.at[pl.ds(sg0, SEGR)],
            out_hbm.at[pl.ds(sg0, SEGR), pl.ds(col0, CW)],
        )
        pltpu.sync_copy(
            max_v.at[pl.ds(sg0, SEGR)],
            out_hbm.at[pl.ds(sg0, SEGR), pl.ds(D + col0, CW)],
        )

    @pl.when(rg == 0)
    def _reduce_lo():
        _reduce(0)

    @pl.when(rg == 1)
    def _reduce_hi():
        _reduce(SEGR)


def kernel(features, batch_indices):
    out, _, _, _ = _seg_meanmax(features, batch_indices.astype(jnp.int32))
    return out


# speedup vs baseline: 1.1557x; 1.1557x over previous
import functools

import jax
import jax.numpy as jnp
from jax import lax
from jax.experimental import pallas as pl
from jax.experimental.pallas import tpu as pltpu
from jax.experimental.pallas import tpu_sc as plsc

N_ROWS = 32768
D = 512
NSEG = 16
LANES = 16
NC = 2
NS = 16
NW = NC * NS
CW = 128
NCB = D // CW
NRG = NW // NCB
RPG = N_ROWS // NRG
CHUNK = 256
NCHUNK = RPG // CHUNK
NPAIR = NCHUNK // 2
NV = CW // LANES
SEGR = NSEG // 2

_mesh = plsc.VectorSubcoreMesh(
    core_axis_name="c", subcore_axis_name="s", num_cores=NC, num_subcores=NS
)


@functools.partial(
    pl.kernel,
    out_type=(
        jax.ShapeDtypeStruct((NSEG, 2 * D), jnp.float32),
        jax.ShapeDtypeStruct((NW, NSEG, CW), jnp.float32),
        jax.ShapeDtypeStruct((NW, NSEG, CW), jnp.float32),
        jax.ShapeDtypeStruct((NW, LANES), jnp.float32),
    ),
    mesh=_mesh,
    scratch_types=[
        pltpu.VMEM((RPG,), jnp.int32),
        pltpu.VMEM((CHUNK, CW), jnp.float32),
        pltpu.VMEM((CHUNK, CW), jnp.float32),
        pltpu.SemaphoreType.DMA,
        pltpu.SemaphoreType.DMA,
        pltpu.VMEM((NSEG, CW), jnp.float32),
        pltpu.VMEM((NSEG, CW), jnp.float32),
        pltpu.VMEM((LANES,), jnp.float32),
        pltpu.VMEM((NRG, LANES), jnp.float32),
        pltpu.VMEM((SEGR, CW), jnp.float32),
        pltpu.VMEM((SEGR, CW), jnp.float32),
    ],
    compiler_params=pltpu.CompilerParams(needs_layout_passes=False),
)
def _seg_meanmax(feat_hbm, idx_hbm, out_hbm, psum_hbm, pmax_hbm, pcnt_hbm,
                 idx_v, fbuf0, fbuf1, sem0, sem1, sum_v, max_v, cnt_v, cbuf,
                 rbuf_s, rbuf_m):
    cid = lax.axis_index("c")
    sid = lax.axis_index("s")
    grp = sid // NRG
    rg = sid % NRG
    cb = cid * (NS // NRG) + grp
    col0 = cb * CW
    rbase = rg * RPG
    wid = cid * NS + sid

    pltpu.sync_copy(idx_hbm.at[pl.ds(rbase, RPG)], idx_v)

    def lower_bound(targets):
        lo = jnp.zeros((LANES,), jnp.int32)
        hi = jnp.full((LANES,), RPG, jnp.int32)

        def step(_, c):
            l, h = c
            mid = lax.div(l + h, 2)
            vals = plsc.load_gather(idx_v, [mid])
            pred = vals < targets
            return (jnp.where(pred, mid + 1, l), jnp.where(pred, h, mid))

        lo, hi = lax.fori_loop(0, 12, step, (lo, hi))
        return lo

    seg_ids = lax.iota(jnp.int32, LANES)
    blo = lower_bound(seg_ids)
    bhi = lower_bound(seg_ids + 1)
    cnt_v[...] = (bhi - blo).astype(jnp.float32)

    zero = jnp.zeros((LANES,), jnp.float32)
    ninf = jnp.full((LANES,), -jnp.inf, jnp.float32)
    for si in range(NSEG):
        for j in range(NV):
            sum_v[si, pl.ds(LANES * j, LANES)] = zero
            max_v[si, pl.ds(LANES * j, LANES)] = ninf

    def copy_of(ci, buf, sem):
        roff = ci * CHUNK
        return pltpu.make_async_copy(
            feat_hbm.at[pl.ds(rbase + roff, CHUNK), pl.ds(col0, CW)], buf, sem
        )

    def process(fbuf, ci):
        roff = ci * CHUNK
        for si in range(NSEG):
            lo = jnp.maximum(blo[si], roff)
            hi = jnp.minimum(bhi[si], roff + CHUNK)

            @pl.when(lo < hi)
            def _nonempty(si=si, lo=lo, hi=hi, roff=roff):
                @plsc.parallel_loop(
                    lo, hi, step=1, unroll=2,
                    carry=((zero,) * NV, (ninf,) * NV),
                )
                def run_loop(i, c):
                    ss, mm = c
                    b = i - roff
                    new_ss = []
                    new_mm = []
                    for j in range(NV):
                        v = fbuf[b, pl.ds(LANES * j, LANES)]
                        new_ss.append(ss[j] + v)
                        new_mm.append(jnp.maximum(mm[j], v))
                    return (tuple(new_ss), tuple(new_mm))

                ss, mm = run_loop
                for j in range(NV):
                    sl = pl.ds(LANES * j, LANES)
                    sum_v[si, sl] = sum_v[si, sl] + ss[j]
                    max_v[si, sl] = jnp.maximum(max_v[si, sl], mm[j])

    copy_of(0, fbuf0, sem0).start()

    def pair_body(q, _):
        c0 = 2 * q
        copy_of(c0, fbuf0, sem0).wait()
        copy_of(c0 + 1, fbuf1, sem1).start()
        process(fbuf0, c0)
        copy_of(c0 + 1, fbuf1, sem1).wait()

        @pl.when(q < NPAIR - 1)
        def _prefetch():
            copy_of(c0 + 2, fbuf0, sem0).start()

        process(fbuf1, c0 + 1)
        return 0

    lax.fori_loop(0, NPAIR, pair_body, 0)

    pltpu.sync_copy(sum_v, psum_hbm.at[wid])
    pltpu.sync_copy(max_v, pmax_hbm.at[wid])
    pltpu.sync_copy(cnt_v, pcnt_hbm.at[wid])
    plsc.subcore_barrier()

    def _reduce(sg0):
        src0 = cid * NS + grp * NRG

        pltpu.sync_copy(pcnt_hbm.at[pl.ds(src0, NRG)], cbuf)
        counts = cbuf[0, :]
        for k in range(1, NRG):
            counts = counts + cbuf[k, :]
        recip = 1.0 / jnp.maximum(counts, 1.0)

        for si in range(SEGR):
            for j in range(NV):
                sl = pl.ds(LANES * j, LANES)
                sum_v[sg0 + si, sl] = zero
                max_v[sg0 + si, sl] = ninf

        def fold(k, _):
            pltpu.sync_copy(psum_hbm.at[src0 + k, pl.ds(sg0, SEGR)], rbuf_s)
            pltpu.sync_copy(pmax_hbm.at[src0 + k, pl.ds(sg0, SEGR)], rbuf_m)
            for si in range(SEGR):
                for j in range(NV):
                    sl = pl.ds(LANES * j, LANES)
                    sum_v[sg0 + si, sl] = sum_v[sg0 + si, sl] + rbuf_s[si, sl]
                    max_v[sg0 + si, sl] = jnp.maximum(
                        max_v[sg0 + si, sl], rbuf_m[si, sl]
                    )
            return 0

        lax.fori_loop(0, NRG, fold, 0)

        for si in range(SEGR):
            seg = sg0 + si
            has = counts[seg] > 0.0
            r = recip[seg]
            for j in range(NV):
                sl = pl.ds(LANES * j, LANES)
                sum_v[seg, sl] = jnp.where(has, sum_v[seg, sl] * r, zero)
                max_v[seg, sl] = jnp.where(has, max_v[seg, sl], zero)

        pltpu.sync_copy(
            sum_v.at[pl.ds(sg0, SEGR)],
            out_hbm.at[pl.ds(sg0, SEGR), pl.ds(col0, CW)],
        )
        pltpu.sync_copy(
            max_v.at[pl.ds(sg0, SEGR)],
            out_hbm.at[pl.ds(sg0, SEGR), pl.ds(D + col0, CW)],
        )

    @pl.when(rg == 0)
    def _reduce_lo():
        _reduce(0)

    @pl.when(rg == 1)
    def _reduce_hi():
        _reduce(SEGR)


def kernel(features, batch_indices):
    out, _, _, _ = _seg_meanmax(features, batch_indices.astype(jnp.int32))
    return out
